# baseline (speedup 1.0000x reference)
import jax
import jax.numpy as jnp
from jax.experimental import pallas as pl
from jax.experimental.pallas import tpu as pltpu

N = 1024
NFEAT = 128
NHID = 64
NHEADS = 3
ALPHA = 0.2
BLK = 256
GRID = N // BLK


def _gat_kernel(x_ref, xb_ref, xt_ref, adj_ref, wcat_ref, wcatb_ref,
                wtflat_ref, a1blkt_ref, a2blk_ref, out_ref,
                hext_ref, nf1_ref, nf2_ref):
    i = pl.program_id(0)

    @pl.when(i == 0)
    def _():
        xv = x_ref[...]
        xb = xb_ref[...]
        xt = xt_ref[...]
        hcat = jnp.dot(xb, wcatb_ref[...],
                       preferred_element_type=jnp.float32).astype(jnp.bfloat16)
        ones = jnp.ones((N, NHID), dtype=jnp.bfloat16)
        for hd in range(NHEADS):
            hext_ref[hd, :, 0:NHID] = hcat[:, hd * NHID:(hd + 1) * NHID]
            hext_ref[hd, :, NHID:2 * NHID] = ones
        c1 = jnp.dot(wcat_ref[...], a1blkt_ref[...],
                     preferred_element_type=jnp.float32)
        nf1_ref[...] = -jnp.dot(xv, c1, preferred_element_type=jnp.float32)
        c2 = jnp.dot(a2blk_ref[...], wtflat_ref[...],
                     preferred_element_type=jnp.float32)
        nf2_ref[...] = -jnp.dot(c2, xt, preferred_element_type=jnp.float32)

    adjb = adj_ref[...]
    rows = jax.lax.broadcasted_iota(jnp.int32, (BLK, N), 0) + i * BLK
    cols = jax.lax.broadcasted_iota(jnp.int32, (BLK, N), 1)
    mask = (adjb != 0.0) | (rows == cols)

    for hd in range(NHEADS):
        nf1b = nf1_ref[pl.ds(i * BLK, BLK), hd:hd + 1]
        nf2r = nf2_ref[hd:hd + 1, :]
        t = nf1b + nf2r
        g = jnp.exp(jnp.minimum(t, ALPHA * t))
        e = jnp.where(mask, g, 0.0).astype(jnp.bfloat16)
        hp = jnp.dot(e, hext_ref[hd], preferred_element_type=jnp.float32)
        v = hp[:, 0:NHID] / hp[:, NHID:NHID + 1]
        out_ref[:, hd * NHID:(hd + 1) * NHID] = jnp.where(
            v > 0.0, v, jnp.exp(jnp.minimum(v, 0.0)) - 1.0)


def kernel(x, adj, W0, a0, W1, a1, W2, a2):
    wcat = jnp.concatenate([W0, W1, W2], axis=1)
    wtflat = jnp.concatenate([W0.T, W1.T, W2.T], axis=0)
    a1blkt = jnp.zeros((NHEADS * NHID, 8), jnp.float32)
    a2blk = jnp.zeros((8, NHEADS * NHID), jnp.float32)
    for hd, a in enumerate((a0, a1, a2)):
        a1blkt = a1blkt.at[hd * NHID:(hd + 1) * NHID, hd].set(a[0, :NHID])
        a2blk = a2blk.at[hd, hd * NHID:(hd + 1) * NHID].set(a[0, NHID:])
    return pl.pallas_call(
        _gat_kernel,
        grid=(GRID,),
        in_specs=[
            pl.BlockSpec((N, NFEAT), lambda i: (0, 0)),
            pl.BlockSpec((N, NFEAT), lambda i: (0, 0)),
            pl.BlockSpec((NFEAT, N), lambda i: (0, 0)),
            pl.BlockSpec((BLK, N), lambda i: (i, 0)),
            pl.BlockSpec((NFEAT, NHEADS * NHID), lambda i: (0, 0)),
            pl.BlockSpec((NFEAT, NHEADS * NHID), lambda i: (0, 0)),
            pl.BlockSpec((NHEADS * NHID, NFEAT), lambda i: (0, 0)),
            pl.BlockSpec((NHEADS * NHID, 8), lambda i: (0, 0)),
            pl.BlockSpec((8, NHEADS * NHID), lambda i: (0, 0)),
        ],
        out_specs=pl.BlockSpec((BLK, NHEADS * NHID), lambda i: (i, 0)),
        out_shape=jax.ShapeDtypeStruct((N, NHEADS * NHID), jnp.float32),
        scratch_shapes=[
            pltpu.VMEM((NHEADS, N, 2 * NHID), jnp.bfloat16),
            pltpu.VMEM((N, 8), jnp.float32),
            pltpu.VMEM((8, N), jnp.float32),
        ],
    )(x, x.astype(jnp.bfloat16), x.T, adj, wcat,
      wcat.astype(jnp.bfloat16), wtflat, a1blkt, a2blk)


# speedup vs baseline: 1.4808x; 1.4808x over previous
import jax
import jax.numpy as jnp
from jax.experimental import pallas as pl
from jax.experimental.pallas import tpu as pltpu

N = 1024
NFEAT = 128
NHID = 64
NHEADS = 3
ALPHA = 0.2
BLK = 256
GRID = N // BLK


def _gat_kernel(x_ref, xt_ref, adj_ref, w_ref, wt_ref, a_ref, out_ref,
                hext_ref, nf1_ref, nf2_ref):
    i = pl.program_id(0)

    @pl.when(i == 0)
    def _():
        xv = x_ref[...]
        xt = xt_ref[...]
        ones = jnp.ones((N, NHID), dtype=jnp.bfloat16)
        for hd in range(NHEADS):
            h = jnp.dot(xv, w_ref[hd], preferred_element_type=jnp.float32)
            hext_ref[hd, :, 0:NHID] = h.astype(jnp.bfloat16)
            hext_ref[hd, :, NHID:2 * NHID] = ones
            a1 = a_ref[hd, 0:NHID]
            a2 = a_ref[hd, NHID:2 * NHID].reshape(1, NHID)
            nf1_ref[hd] = -jnp.sum(h * a1[None, :], axis=1, keepdims=True)
            c2 = jnp.dot(a2, wt_ref[hd], preferred_element_type=jnp.float32)
            nf2_ref[hd] = -jnp.dot(c2, xt, preferred_element_type=jnp.float32)

    adjb = adj_ref[...]
    rows = jax.lax.broadcasted_iota(jnp.int32, (BLK, N), 0) + i * BLK
    cols = jax.lax.broadcasted_iota(jnp.int32, (BLK, N), 1)
    mask = (adjb != 0.0) | (rows == cols)

    for hd in range(NHEADS):
        nf1b = nf1_ref[hd, pl.ds(i * BLK, BLK), :]
        nf2r = nf2_ref[hd]
        t = nf1b + nf2r
        g = jnp.exp(jnp.minimum(t, ALPHA * t))
        e = jnp.where(mask, g, 0.0).astype(jnp.bfloat16)
        hp = jnp.dot(e, hext_ref[hd], preferred_element_type=jnp.float32)
        v = hp[:, 0:NHID] / hp[:, NHID:NHID + 1]
        out_ref[:, hd * NHID:(hd + 1) * NHID] = jnp.where(
            v > 0.0, v, jnp.exp(jnp.minimum(v, 0.0)) - 1.0)


def kernel(x, adj, W0, a0, W1, a1, W2, a2):
    W = jnp.stack([W0, W1, W2])
    Wt = jnp.transpose(W, (0, 2, 1))
    A = jnp.stack([a0[0], a1[0], a2[0]])
    return pl.pallas_call(
        _gat_kernel,
        grid=(GRID,),
        in_specs=[
            pl.BlockSpec((N, NFEAT), lambda i: (0, 0)),
            pl.BlockSpec((NFEAT, N), lambda i: (0, 0)),
            pl.BlockSpec((BLK, N), lambda i: (i, 0)),
            pl.BlockSpec((NHEADS, NFEAT, NHID), lambda i: (0, 0, 0)),
            pl.BlockSpec((NHEADS, NHID, NFEAT), lambda i: (0, 0, 0)),
            pl.BlockSpec((NHEADS, 2 * NHID), lambda i: (0, 0)),
        ],
        out_specs=pl.BlockSpec((BLK, NHEADS * NHID), lambda i: (i, 0)),
        out_shape=jax.ShapeDtypeStruct((N, NHEADS * NHID), jnp.float32),
        scratch_shapes=[
            pltpu.VMEM((NHEADS, N, 2 * NHID), jnp.bfloat16),
            pltpu.VMEM((NHEADS, N, 1), jnp.float32),
            pltpu.VMEM((NHEADS, 1, N), jnp.float32),
        ],
    )(x, x.T, adj, W, Wt, A)


# speedup vs baseline: 1.5571x; 1.0516x over previous
import jax
import jax.numpy as jnp
from jax.experimental import pallas as pl
from jax.experimental.pallas import tpu as pltpu

N = 1024
NFEAT = 128
NHID = 64
NHEADS = 3
ALPHA = 0.2
BLK = 256
GRID = N // BLK


def _gat_kernel(x_ref, adj_ref, w0_ref, a0_ref, w1_ref, a1_ref, w2_ref, a2_ref,
                out_ref, hext_ref, nf1_ref, nf2_ref):
    i = pl.program_id(0)
    w_refs = (w0_ref, w1_ref, w2_ref)
    a_refs = (a0_ref, a1_ref, a2_ref)

    @pl.when(i == 0)
    def _():
        xv = x_ref[...]
        ones = jnp.ones((N, NHID), dtype=jnp.bfloat16)
        for hd in range(NHEADS):
            w = w_refs[hd][...]
            h = jnp.dot(xv, w, preferred_element_type=jnp.float32)
            hext_ref[hd, :, 0:NHID] = h.astype(jnp.bfloat16)
            hext_ref[hd, :, NHID:2 * NHID] = ones
            a1 = a_refs[hd][0, 0:NHID]
            a2 = a_refs[hd][:, NHID:2 * NHID]
            nf1_ref[hd] = -jnp.sum(h * a1[None, :], axis=1, keepdims=True)
            ht = jax.lax.dot_general(
                w, xv, (((0,), (1,)), ((), ())),
                preferred_element_type=jnp.float32)
            nf2_ref[hd] = -jnp.dot(a2, ht, preferred_element_type=jnp.float32)

    adjb = adj_ref[...]
    rows = jax.lax.broadcasted_iota(jnp.int32, (BLK, N), 0) + i * BLK
    cols = jax.lax.broadcasted_iota(jnp.int32, (BLK, N), 1)
    mask = (adjb != 0.0) | (rows == cols)

    for hd in range(NHEADS):
        nf1b = nf1_ref[hd, pl.ds(i * BLK, BLK), :]
        nf2r = nf2_ref[hd]
        t = nf1b + nf2r
        g = jnp.exp(jnp.minimum(t, ALPHA * t))
        e = jnp.where(mask, g, 0.0).astype(jnp.bfloat16)
        hp = jnp.dot(e, hext_ref[hd], preferred_element_type=jnp.float32)
        v = hp[:, 0:NHID] / hp[:, NHID:NHID + 1]
        out_ref[:, hd * NHID:(hd + 1) * NHID] = jnp.where(
            v > 0.0, v, jnp.exp(jnp.minimum(v, 0.0)) - 1.0)


def kernel(x, adj, W0, a0, W1, a1, W2, a2):
    full = lambda shape: pl.BlockSpec(shape, lambda i: tuple(0 for _ in shape))
    wspec = full((NFEAT, NHID))
    aspec = full((1, 2 * NHID))
    return pl.pallas_call(
        _gat_kernel,
        grid=(GRID,),
        in_specs=[
            full((N, NFEAT)),
            pl.BlockSpec((BLK, N), lambda i: (i, 0)),
            wspec, aspec, wspec, aspec, wspec, aspec,
        ],
        out_specs=pl.BlockSpec((BLK, NHEADS * NHID), lambda i: (i, 0)),
        out_shape=jax.ShapeDtypeStruct((N, NHEADS * NHID), jnp.float32),
        scratch_shapes=[
            pltpu.VMEM((NHEADS, N, 2 * NHID), jnp.bfloat16),
            pltpu.VMEM((NHEADS, N, 1), jnp.float32),
            pltpu.VMEM((NHEADS, 1, N), jnp.float32),
        ],
    )(x, adj, W0, a0, W1, a1, W2, a2)


# speedup vs baseline: 1.5748x; 1.0114x over previous
import jax
import jax.numpy as jnp
from jax.experimental import pallas as pl
from jax.experimental.pallas import tpu as pltpu

N = 1024
NFEAT = 128
NHID = 64
NHEADS = 3
ALPHA = 0.2
LOG2E = 1.4426950408889634
BLK = 256
GRID = N // BLK


def _gat_kernel(x_ref, adj_ref, w0_ref, a0_ref, w1_ref, a1_ref, w2_ref, a2_ref,
                out_ref, hext_ref, nf1_ref, nf2_ref):
    i = pl.program_id(0)
    w_refs = (w0_ref, w1_ref, w2_ref)
    a_refs = (a0_ref, a1_ref, a2_ref)

    @pl.when(i == 0)
    def _():
        xv = x_ref[...]
        ones = jnp.ones((N, NHID), dtype=jnp.bfloat16)
        for hd in range(NHEADS):
            w = w_refs[hd][...]
            h = jnp.dot(xv, w, preferred_element_type=jnp.float32)
            hext_ref[hd, :, 0:NHID] = h.astype(jnp.bfloat16)
            hext_ref[hd, :, NHID:2 * NHID] = ones
            a1 = a_refs[hd][:, 0:NHID] * (-LOG2E)
            a2 = a_refs[hd][:, NHID:2 * NHID] * (-LOG2E)
            nf1_ref[hd] = jax.lax.dot_general(
                h, a1, (((1,), (1,)), ((), ())),
                preferred_element_type=jnp.float32)
            ht = jax.lax.dot_general(
                w, xv, (((0,), (1,)), ((), ())),
                preferred_element_type=jnp.float32)
            nf2_ref[hd] = jnp.dot(a2, ht, preferred_element_type=jnp.float32)

    adjb = adj_ref[...]
    rows = jax.lax.broadcasted_iota(jnp.int32, (BLK, N), 0) + i * BLK
    cols = jax.lax.broadcasted_iota(jnp.int32, (BLK, N), 1)
    mask = (adjb != 0.0) | (rows == cols)

    for hd in range(NHEADS):
        nf1b = nf1_ref[hd, pl.ds(i * BLK, BLK), :]
        nf2r = nf2_ref[hd]
        t = nf1b + nf2r
        g = jnp.exp2(jnp.minimum(t, ALPHA * t))
        e = jnp.where(mask, g, 0.0).astype(jnp.bfloat16)
        hp = jnp.dot(e, hext_ref[hd], preferred_element_type=jnp.float32)
        v = hp[:, 0:NHID] / hp[:, NHID:NHID + 1]
        out_ref[:, hd * NHID:(hd + 1) * NHID] = jnp.where(
            v > 0.0, v, jnp.exp(jnp.minimum(v, 0.0)) - 1.0)


def kernel(x, adj, W0, a0, W1, a1, W2, a2):
    full = lambda shape: pl.BlockSpec(shape, lambda i: tuple(0 for _ in shape))
    wspec = full((NFEAT, NHID))
    aspec = full((1, 2 * NHID))
    return pl.pallas_call(
        _gat_kernel,
        grid=(GRID,),
        in_specs=[
            full((N, NFEAT)),
            pl.BlockSpec((BLK, N), lambda i: (i, 0)),
            wspec, aspec, wspec, aspec, wspec, aspec,
        ],
        out_specs=pl.BlockSpec((BLK, NHEADS * NHID), lambda i: (i, 0)),
        out_shape=jax.ShapeDtypeStruct((N, NHEADS * NHID), jnp.float32),
        scratch_shapes=[
            pltpu.VMEM((NHEADS, N, 2 * NHID), jnp.bfloat16),
            pltpu.VMEM((NHEADS, N, 1), jnp.float32),
            pltpu.VMEM((NHEADS, 1, N), jnp.float32),
        ],
    )(x, adj, W0, a0, W1, a1, W2, a2)


# speedup vs baseline: 1.6668x; 1.0584x over previous
import jax
import jax.numpy as jnp
from jax.experimental import pallas as pl
from jax.experimental.pallas import tpu as pltpu

N = 1024
NFEAT = 128
NHID = 64
NHEADS = 3
ALPHA = 0.2
LOG2E = 1.4426950408889634
BLK = 512
GRID = N // BLK


def _gat_kernel(x_ref, adj_ref, w0_ref, a0_ref, w1_ref, a1_ref, w2_ref, a2_ref,
                out_ref, hext_ref, nf1_ref, nf2_ref):
    i = pl.program_id(0)
    w_refs = (w0_ref, w1_ref, w2_ref)
    a_refs = (a0_ref, a1_ref, a2_ref)

    @pl.when(i == 0)
    def _():
        xv = x_ref[...]
        ones = jnp.ones((N, NHID), dtype=jnp.bfloat16)
        for hd in range(NHEADS):
            w = w_refs[hd][...]
            h = jnp.dot(xv, w, preferred_element_type=jnp.float32)
            hext_ref[hd, :, 0:NHID] = h.astype(jnp.bfloat16)
            hext_ref[hd, :, NHID:2 * NHID] = ones
            a1 = a_refs[hd][:, 0:NHID] * (-LOG2E)
            a2 = a_refs[hd][:, NHID:2 * NHID] * (-LOG2E)
            nf1_ref[hd] = jax.lax.dot_general(
                h, a1, (((1,), (1,)), ((), ())),
                preferred_element_type=jnp.float32)
            ht = jax.lax.dot_general(
                w, xv, (((0,), (1,)), ((), ())),
                preferred_element_type=jnp.float32)
            nf2_ref[hd] = jnp.dot(a2, ht, preferred_element_type=jnp.float32)

    adjb = adj_ref[...]
    rows = jax.lax.broadcasted_iota(jnp.int32, (BLK, N), 0) + i * BLK
    cols = jax.lax.broadcasted_iota(jnp.int32, (BLK, N), 1)
    mask = (adjb != 0.0) | (rows == cols)

    for hd in range(NHEADS):
        nf1b = nf1_ref[hd, pl.ds(i * BLK, BLK), :]
        nf2r = nf2_ref[hd]
        t = nf1b + nf2r
        g = jnp.exp2(jnp.minimum(t, ALPHA * t))
        e = jnp.where(mask, g, 0.0).astype(jnp.bfloat16)
        hp = jnp.dot(e, hext_ref[hd], preferred_element_type=jnp.float32)
        v = hp[:, 0:NHID] / hp[:, NHID:NHID + 1]
        out_ref[:, hd * NHID:(hd + 1) * NHID] = jnp.where(
            v > 0.0, v, jnp.exp(jnp.minimum(v, 0.0)) - 1.0)


def kernel(x, adj, W0, a0, W1, a1, W2, a2):
    full = lambda shape: pl.BlockSpec(shape, lambda i: tuple(0 for _ in shape))
    wspec = full((NFEAT, NHID))
    aspec = full((1, 2 * NHID))
    return pl.pallas_call(
        _gat_kernel,
        grid=(GRID,),
        in_specs=[
            full((N, NFEAT)),
            pl.BlockSpec((BLK, N), lambda i: (i, 0)),
            wspec, aspec, wspec, aspec, wspec, aspec,
        ],
        out_specs=pl.BlockSpec((BLK, NHEADS * NHID), lambda i: (i, 0)),
        out_shape=jax.ShapeDtypeStruct((N, NHEADS * NHID), jnp.float32),
        scratch_shapes=[
            pltpu.VMEM((NHEADS, N, 2 * NHID), jnp.bfloat16),
            pltpu.VMEM((NHEADS, N, 1), jnp.float32),
            pltpu.VMEM((NHEADS, 1, N), jnp.float32),
        ],
    )(x, adj, W0, a0, W1, a1, W2, a2)
